# Initial kernel scaffold; baseline (speedup 1.0000x reference)
import jax
import jax.numpy as jnp
from jax.experimental import pallas as pl


def kernel(x, emb, W1, b1, W2, b2, A1, a1b, A2, a2b, A3, a3b, C1, c1b, C2, c2b):
    raise NotImplementedError("write your pallas kernel here")


# speedup vs baseline: 1.3906x; 1.3906x over previous
import functools

import jax
import jax.numpy as jnp
from jax import lax
from jax.experimental import pallas as pl
from jax.experimental.pallas import tpu as pltpu
from jax.experimental.pallas import tpu_sc as plsc

_B, _L, _D, _V = 32, 2048, 1024, 100000
_K = max(1, int(_L * 0.1))
_N = _B * _L
_T = 512
_C = 32


def _sc_gather(emb, xflat):
    info = plsc.get_sparse_core_info()
    nc, ns = info.num_cores, info.num_subcores
    nw = nc * ns
    n = xflat.shape[0]
    d = emb.shape[1]
    per_w = n // nw
    n_chunks = per_w // _C

    mesh = plsc.VectorSubcoreMesh(core_axis_name="c", subcore_axis_name="s")

    @functools.partial(
        pl.kernel,
        mesh=mesh,
        out_type=jax.ShapeDtypeStruct((n, d), jnp.float32),
        scratch_types=[
            pltpu.VMEM((_C,), jnp.int32),
            pltpu.VMEM((_C,), jnp.int32),
            pltpu.VMEM((_C, d), jnp.float32),
            pltpu.VMEM((_C, d), jnp.float32),
            pltpu.SemaphoreType.DMA,
            pltpu.SemaphoreType.DMA,
            pltpu.SemaphoreType.DMA,
            pltpu.SemaphoreType.DMA,
        ],
    )
    def k(emb_h, idx_h, out_h, idx0, idx1, rows0, rows1, sg0, sg1, ss0, ss1):
        wid = lax.axis_index("s") * nc + lax.axis_index("c")
        base = wid * per_w
        idxv, rowsv, sg, ss = (idx0, idx1), (rows0, rows1), (sg0, sg1), (ss0, ss1)

        pltpu.sync_copy(idx_h.at[pl.ds(base, _C)], idx0)
        pltpu.make_async_copy(emb_h.at[idx0], rows0, sg0).start()

        def outer(o, carry):
            for b in (0, 1):
                g = 2 * o + b
                ob = 1 - b

                @pl.when(g >= 1)
                def _drain():
                    pltpu.make_async_copy(
                        rowsv[ob], out_h.at[pl.ds(base + (g - 1) * _C, _C)],
                        ss[ob]).wait()

                @pl.when(g + 1 < n_chunks)
                def _prefetch():
                    pltpu.sync_copy(
                        idx_h.at[pl.ds(base + (g + 1) * _C, _C)], idxv[ob])
                    pltpu.make_async_copy(
                        emb_h.at[idxv[ob]], rowsv[ob], sg[ob]).start()

                pltpu.make_async_copy(emb_h.at[idxv[b]], rowsv[b], sg[b]).wait()
                pltpu.make_async_copy(
                    rowsv[b], out_h.at[pl.ds(base + g * _C, _C)], ss[b]).start()
            return carry

        lax.fori_loop(0, n_chunks // 2, outer, 0)

        pltpu.make_async_copy(
            rows1, out_h.at[pl.ds(base + (n_chunks - 1) * _C, _C)], ss1).wait()

    return k(emb, xflat)


def _big_body(e_ref, W1_ref, b1_ref, W2T_ref, A1T_ref,
              sT_ref, t8_ref, esum_ref):
    i = pl.program_id(0)
    et = e_ref[...]
    etb = et.astype(jnp.bfloat16)
    h = jnp.maximum(
        lax.dot(etb, W1_ref[...], preferred_element_type=jnp.float32)
        + b1_ref[...], 0.0)
    sT_ref[...] = lax.dot_general(
        W2T_ref[...], h.astype(jnp.bfloat16), (((1,), (1,)), ((), ())),
        preferred_element_type=jnp.float32)[None]
    a1kvT = (A1T_ref[...][:, _D:2 * _D]
             + A1T_ref[...][:, 2 * _D:]).astype(jnp.bfloat16)
    t8_ref[...] = lax.dot_general(
        a1kvT, etb, (((1,), (1,)), ((), ())),
        preferred_element_type=jnp.float32)
    part = jnp.sum(et, axis=0, keepdims=True)[None]

    @pl.when(i % (_L // _T) == 0)
    def _init():
        esum_ref[...] = part

    @pl.when(i % (_L // _T) != 0)
    def _acc():
        esum_ref[...] = esum_ref[...] + part


def _tc_big(e, W1b, b1r, W2Tb, A1T):
    grid = _N // _T
    lpt = _L // _T
    return pl.pallas_call(
        _big_body,
        grid=(grid,),
        in_specs=[
            pl.BlockSpec((_T, _D), lambda i: (i, 0)),
            pl.BlockSpec((_D, _D // 2), lambda i: (0, 0)),
            pl.BlockSpec((1, _D // 2), lambda i: (0, 0)),
            pl.BlockSpec((1, _D // 2), lambda i: (0, 0)),
            pl.BlockSpec((8, 3 * _D), lambda i: (0, 0)),
        ],
        out_specs=[
            pl.BlockSpec((1, 1, _T), lambda i: (i, 0, 0)),
            pl.BlockSpec((8, _T), lambda i: (0, i)),
            pl.BlockSpec((1, 1, _D), lambda i: (i // lpt, 0, 0)),
        ],
        out_shape=[
            jax.ShapeDtypeStruct((grid, 1, _T), jnp.float32),
            jax.ShapeDtypeStruct((8, _N), jnp.float32),
            jax.ShapeDtypeStruct((_B, 1, _D), jnp.float32),
        ],
        compiler_params=pltpu.CompilerParams(
            dimension_semantics=("arbitrary",)),
    )(e, W1b, b1r, W2Tb, A1T)


def _small_body(s_ref, t8_ref, esum_ref, A1T_ref, a1b_ref, A2_ref, a2b_ref,
                A3_ref, a3b_ref, C1_ref, c1b_ref, C2_ref, c2b_ref, out_ref):
    s = s_ref[...]
    bits = lax.bitcast_convert_type(s, jnp.int32)
    key = jnp.where(bits >= 0, bits, bits ^ jnp.int32(0x7FFFFFFF))
    msb = jnp.int32(-2147483648)

    def bit_step(it, p):
        cand = p | lax.shift_left(jnp.int32(1), 31 - it)
        cnt = jnp.sum((key >= (cand ^ msb)).astype(jnp.int32),
                      axis=1, keepdims=True)
        return jnp.where(cnt >= _K, cand, p)

    p = lax.fori_loop(0, 32, bit_step,
                      jnp.zeros((_B, 1), jnp.int32), unroll=True)
    theta = p ^ msb
    gt = (key > theta).astype(jnp.float32)
    eq = (key == theta).astype(jnp.float32)
    cnt_gt = jnp.sum(gt, axis=1, keepdims=True)
    cnt_eq = jnp.sum(eq, axis=1, keepdims=True)
    w = gt + eq * ((_K - cnt_gt) / cnt_eq)

    nrow = lax.broadcasted_iota(jnp.int32, (_N, _B), 0) // _L
    bcol = lax.broadcasted_iota(jnp.int32, (_N, _B), 1)
    E = (nrow == bcol).astype(jnp.bfloat16)

    a1qT = A1T_ref[...][:, :_D]
    q8T = lax.dot_general(
        a1qT, esum_ref[...] * (1.0 / _L), (((1,), (1,)), ((), ())),
        preferred_element_type=jnp.float32)
    qfull = lax.dot_general(
        q8T.astype(jnp.bfloat16), E, (((1,), (1,)), ((), ())),
        preferred_element_type=jnp.float32)

    m1 = jnp.maximum(t8_ref[...] + qfull + a1b_ref[...], 0.0)
    m2 = jnp.maximum(
        lax.dot_general(A2_ref[...], m1, (((0,), (0,)), ((), ())),
                        preferred_element_type=jnp.float32)
        + a2b_ref[...], 0.0)

    wflat = jnp.reshape(w, (1, _N)).astype(jnp.bfloat16)
    m2w = m2.astype(jnp.bfloat16) * wflat
    out8T = lax.dot(m2w, E, preferred_element_type=jnp.float32) * (1.0 / _K)

    outp = lax.dot_general(out8T, A3_ref[...], (((0,), (0,)), ((), ())),
                           preferred_element_type=jnp.float32) \
        + a3b_ref[...]
    c = jnp.maximum(
        lax.dot(outp, C1_ref[...], preferred_element_type=jnp.float32)
        + c1b_ref[...], 0.0)
    z = lax.dot(c, C2_ref[...], preferred_element_type=jnp.float32) \
        + c2b_ref[...]
    out_ref[...] = jax.nn.sigmoid(z)


def _tc_small(sT, t8, esum, A1T, a1bc, A2, a2bc, A3, a3br, C1, c1br, C2, c2bs):
    return pl.pallas_call(
        _small_body,
        out_shape=jax.ShapeDtypeStruct((_B, 1), jnp.float32),
    )(sT, t8, esum, A1T, a1bc, A2, a2bc, A3, a3br, C1, c1br, C2, c2bs)


def kernel(x, emb, W1, b1, W2, b2, A1, a1b, A2, a2b, A3, a3b, C1, c1b, C2, c2b):
    xflat = x.reshape(-1).astype(jnp.int32)
    e = _sc_gather(emb, xflat)
    sT3, t8, esum3 = _tc_big(
        e,
        W1.astype(jnp.bfloat16),
        b1.reshape(1, -1),
        W2.T.astype(jnp.bfloat16),
        A1.T,
    )
    pred = _tc_small(
        sT3.reshape(_B, _L), t8, esum3.reshape(_B, _D),
        A1.T,
        a1b.reshape(-1, 1),
        A2,
        a2b.reshape(-1, 1),
        A3,
        a3b.reshape(1, -1),
        C1,
        c1b.reshape(1, -1),
        C2,
        c2b.reshape(1, 1),
    )
    return pred[:, 0]


# speedup vs baseline: 1.5651x; 1.1255x over previous
import functools

import jax
import jax.numpy as jnp
from jax import lax
from jax.experimental import pallas as pl
from jax.experimental.pallas import tpu as pltpu
from jax.experimental.pallas import tpu_sc as plsc

_B, _L, _D, _V = 32, 2048, 1024, 100000
_K = max(1, int(_L * 0.1))
_N = _B * _L
_T = 512
_C = 32
_NCH = 4


def _sc_gather(emb, xflat):
    info = plsc.get_sparse_core_info()
    nc, ns = info.num_cores, info.num_subcores
    nw = nc * ns
    n = xflat.shape[0]
    d = emb.shape[1]
    per_w = n // nw
    n_chunks = per_w // _C

    mesh = plsc.VectorSubcoreMesh(core_axis_name="c", subcore_axis_name="s")

    @functools.partial(
        pl.kernel,
        mesh=mesh,
        out_type=jax.ShapeDtypeStruct((n, d), jnp.float32),
        scratch_types=[
            pltpu.VMEM((_C,), jnp.int32),
            pltpu.VMEM((_C,), jnp.int32),
            pltpu.VMEM((_C, d), jnp.float32),
            pltpu.VMEM((_C, d), jnp.float32),
            pltpu.SemaphoreType.DMA,
            pltpu.SemaphoreType.DMA,
            pltpu.SemaphoreType.DMA,
            pltpu.SemaphoreType.DMA,
        ],
    )
    def k(emb_h, idx_h, out_h, idx0, idx1, rows0, rows1, sg0, sg1, ss0, ss1):
        wid = lax.axis_index("s") * nc + lax.axis_index("c")
        base = wid * per_w
        idxv, rowsv, sg, ss = (idx0, idx1), (rows0, rows1), (sg0, sg1), (ss0, ss1)

        pltpu.sync_copy(idx_h.at[pl.ds(base, _C)], idx0)
        pltpu.make_async_copy(emb_h.at[idx0], rows0, sg0).start()

        def outer(o, carry):
            for b in (0, 1):
                g = 2 * o + b
                ob = 1 - b

                @pl.when(g >= 1)
                def _drain():
                    pltpu.make_async_copy(
                        rowsv[ob], out_h.at[pl.ds(base + (g - 1) * _C, _C)],
                        ss[ob]).wait()

                @pl.when(g + 1 < n_chunks)
                def _prefetch():
                    pltpu.sync_copy(
                        idx_h.at[pl.ds(base + (g + 1) * _C, _C)], idxv[ob])
                    pltpu.make_async_copy(
                        emb_h.at[idxv[ob]], rowsv[ob], sg[ob]).start()

                pltpu.make_async_copy(emb_h.at[idxv[b]], rowsv[b], sg[b]).wait()
                pltpu.make_async_copy(
                    rowsv[b], out_h.at[pl.ds(base + g * _C, _C)], ss[b]).start()
            return carry

        lax.fori_loop(0, n_chunks // 2, outer, 0)

        pltpu.make_async_copy(
            rows1, out_h.at[pl.ds(base + (n_chunks - 1) * _C, _C)], ss1).wait()

    return k(emb, xflat)


def _big_body(e_ref, W1_ref, b1_ref, W2T_ref, A1T_ref,
              sT_ref, t8_ref, esum_ref):
    i = pl.program_id(0)
    et = e_ref[...]
    etb = et.astype(jnp.bfloat16)
    h = jnp.maximum(
        lax.dot(etb, W1_ref[...], preferred_element_type=jnp.float32)
        + b1_ref[...], 0.0)
    sT_ref[...] = lax.dot_general(
        W2T_ref[...], h.astype(jnp.bfloat16), (((1,), (1,)), ((), ())),
        preferred_element_type=jnp.float32)[None]
    a1kvT = (A1T_ref[...][:, _D:2 * _D]
             + A1T_ref[...][:, 2 * _D:]).astype(jnp.bfloat16)
    t8_ref[...] = lax.dot_general(
        a1kvT, etb, (((1,), (1,)), ((), ())),
        preferred_element_type=jnp.float32)
    part = jnp.sum(et, axis=0, keepdims=True)[None]

    @pl.when(i % (_L // _T) == 0)
    def _init():
        esum_ref[...] = part

    @pl.when(i % (_L // _T) != 0)
    def _acc():
        esum_ref[...] = esum_ref[...] + part


def _tc_big(e, W1b, b1r, W2Tb, A1T):
    n = e.shape[0]
    grid = n // _T
    lpt = _L // _T
    return pl.pallas_call(
        _big_body,
        grid=(grid,),
        in_specs=[
            pl.BlockSpec((_T, _D), lambda i: (i, 0)),
            pl.BlockSpec((_D, _D // 2), lambda i: (0, 0)),
            pl.BlockSpec((1, _D // 2), lambda i: (0, 0)),
            pl.BlockSpec((1, _D // 2), lambda i: (0, 0)),
            pl.BlockSpec((8, 3 * _D), lambda i: (0, 0)),
        ],
        out_specs=[
            pl.BlockSpec((1, 1, _T), lambda i: (i, 0, 0)),
            pl.BlockSpec((8, _T), lambda i: (0, i)),
            pl.BlockSpec((1, 1, _D), lambda i: (i // lpt, 0, 0)),
        ],
        out_shape=[
            jax.ShapeDtypeStruct((grid, 1, _T), jnp.float32),
            jax.ShapeDtypeStruct((8, n), jnp.float32),
            jax.ShapeDtypeStruct((n // _L, 1, _D), jnp.float32),
        ],
        compiler_params=pltpu.CompilerParams(
            dimension_semantics=("arbitrary",)),
    )(e, W1b, b1r, W2Tb, A1T)


def _small_body(s_ref, t8_ref, esum_ref, A1T_ref, a1b_ref, A2_ref, a2b_ref,
                A3_ref, a3b_ref, C1_ref, c1b_ref, C2_ref, c2b_ref, out_ref):
    s = s_ref[...]
    bits = lax.bitcast_convert_type(s, jnp.int32)
    key = jnp.where(bits >= 0, bits, bits ^ jnp.int32(0x7FFFFFFF))
    msb = jnp.int32(-2147483648)

    def bit_step(it, p):
        cand = p | lax.shift_left(jnp.int32(1), 31 - it)
        cnt = jnp.sum((key >= (cand ^ msb)).astype(jnp.int32),
                      axis=1, keepdims=True)
        return jnp.where(cnt >= _K, cand, p)

    p = lax.fori_loop(0, 32, bit_step,
                      jnp.zeros((_B, 1), jnp.int32), unroll=True)
    theta = p ^ msb
    gt = (key > theta).astype(jnp.float32)
    eq = (key == theta).astype(jnp.float32)
    cnt_gt = jnp.sum(gt, axis=1, keepdims=True)
    cnt_eq = jnp.sum(eq, axis=1, keepdims=True)
    w = gt + eq * ((_K - cnt_gt) / cnt_eq)

    nrow = lax.broadcasted_iota(jnp.int32, (_N, _B), 0) // _L
    bcol = lax.broadcasted_iota(jnp.int32, (_N, _B), 1)
    E = (nrow == bcol).astype(jnp.bfloat16)

    a1qT = A1T_ref[...][:, :_D]
    q8T = lax.dot_general(
        a1qT, esum_ref[...] * (1.0 / _L), (((1,), (1,)), ((), ())),
        preferred_element_type=jnp.float32)
    qfull = lax.dot_general(
        q8T.astype(jnp.bfloat16), E, (((1,), (1,)), ((), ())),
        preferred_element_type=jnp.float32)

    m1 = jnp.maximum(t8_ref[...] + qfull + a1b_ref[...], 0.0)
    m2 = jnp.maximum(
        lax.dot_general(A2_ref[...], m1, (((0,), (0,)), ((), ())),
                        preferred_element_type=jnp.float32)
        + a2b_ref[...], 0.0)

    wflat = jnp.reshape(w, (1, _N)).astype(jnp.bfloat16)
    m2w = m2.astype(jnp.bfloat16) * wflat
    out8T = lax.dot(m2w, E, preferred_element_type=jnp.float32) * (1.0 / _K)

    outp = lax.dot_general(out8T, A3_ref[...], (((0,), (0,)), ((), ())),
                           preferred_element_type=jnp.float32) \
        + a3b_ref[...]
    c = jnp.maximum(
        lax.dot(outp, C1_ref[...], preferred_element_type=jnp.float32)
        + c1b_ref[...], 0.0)
    z = lax.dot(c, C2_ref[...], preferred_element_type=jnp.float32) \
        + c2b_ref[...]
    out_ref[...] = jax.nn.sigmoid(z)


def _tc_small(sT, t8, esum, A1T, a1bc, A2, a2bc, A3, a3br, C1, c1br, C2, c2bs):
    return pl.pallas_call(
        _small_body,
        out_shape=jax.ShapeDtypeStruct((_B, 1), jnp.float32),
    )(sT, t8, esum, A1T, a1bc, A2, a2bc, A3, a3br, C1, c1br, C2, c2bs)


def kernel(x, emb, W1, b1, W2, b2, A1, a1b, A2, a2b, A3, a3b, C1, c1b, C2, c2b):
    xflat = x.reshape(-1).astype(jnp.int32)
    W1b = W1.astype(jnp.bfloat16)
    b1r = b1.reshape(1, -1)
    W2Tb = W2.T.astype(jnp.bfloat16)
    A1T = A1.T
    nc = _N // _NCH
    sTs, t8s, esums = [], [], []
    for c in range(_NCH):
        e_c = _sc_gather(emb, lax.dynamic_slice_in_dim(xflat, c * nc, nc))
        sT3c, t8c, esum3c = _tc_big(e_c, W1b, b1r, W2Tb, A1T)
        sTs.append(sT3c)
        t8s.append(t8c)
        esums.append(esum3c)
    sT3 = jnp.concatenate(sTs, axis=0)
    t8 = jnp.concatenate(t8s, axis=1)
    esum3 = jnp.concatenate(esums, axis=0)
    pred = _tc_small(
        sT3.reshape(_B, _L), t8, esum3.reshape(_B, _D),
        A1.T,
        a1b.reshape(-1, 1),
        A2,
        a2b.reshape(-1, 1),
        A3,
        a3b.reshape(1, -1),
        C1,
        c1b.reshape(1, -1),
        C2,
        c2b.reshape(1, 1),
    )
    return pred[:, 0]
